# baseline (speedup 1.0000x reference)
import functools

import jax
import jax.numpy as jnp
from jax.experimental import pallas as pl
from jax.experimental.pallas import tpu as pltpu

_ROWS = 4
_NJ = 2


def _round_up(v, m):
    return (v + m - 1) // m * m


def _conv3_kernel(w_ref, x_ref, o_ref, *, rows, nj):
    j = pl.program_id(1)
    l = x_ref.shape[2]
    t = l // nj
    base = j * t
    for r in range(rows):
        xc = x_ref[r, :, pl.ds(base, t)].astype(jnp.bfloat16)
        c = xc.shape[0]
        tb = t // 128
        lchunk = x_ref[r, :, pl.ds(jnp.maximum(j * tb - 1, 0) * 128, 128)]
        lcol = jnp.where(j == 0, 0.0, lchunk[:, 127:128]).astype(jnp.bfloat16)
        rchunk = x_ref[r, :, pl.ds(jnp.minimum((j + 1) * tb, l // 128 - 1) * 128, 128)]
        rcol = jnp.where(j == nj - 1, 0.0, rchunk[:, 0:1]).astype(jnp.bfloat16)
        x3 = jnp.concatenate(
            [jnp.concatenate([lcol, xc[:, : t - 1]], axis=1),
             xc,
             jnp.concatenate([xc[:, 1:], rcol], axis=1)], axis=0)
        acc = jnp.dot(w_ref[...], x3, preferred_element_type=jnp.float32)
        o_ref[r] = jnp.maximum(acc, 0.0)


@jax.jit
def kernel(x, weight):
    N, C_in, L = x.shape
    C_out, C_in_w, K = weight.shape
    assert C_in_w == C_in and K == 3
    L_out = L

    C_in_pad = _round_up(C_in, 8)
    C_out_pad = _round_up(C_out, 8)
    L_pad = _round_up(L, 128 * _NJ)
    xp = jnp.pad(x, ((0, 0), (0, C_in_pad - C_in), (0, L_pad - L)))
    w3 = jnp.transpose(weight, (2, 0, 1)).astype(jnp.bfloat16)
    w3 = jnp.pad(w3, ((0, 0), (0, C_out_pad - C_out), (0, C_in_pad - C_in)))
    w3 = jnp.transpose(w3, (1, 0, 2)).reshape(C_out_pad, K * C_in_pad)

    rows = _ROWS if N % _ROWS == 0 else 1
    out = pl.pallas_call(
        functools.partial(_conv3_kernel, rows=rows, nj=_NJ),
        out_shape=jax.ShapeDtypeStruct((N, C_out_pad, L_pad), x.dtype),
        grid=(N // rows, _NJ),
        in_specs=[
            pl.BlockSpec((C_out_pad, K * C_in_pad), lambda n, j: (0, 0)),
            pl.BlockSpec((rows, C_in_pad, L_pad), lambda n, j: (n, 0, 0)),
        ],
        out_specs=pl.BlockSpec((rows, C_out_pad, L_pad // _NJ),
                               lambda n, j: (n, 0, j)),
        compiler_params=pltpu.CompilerParams(
            dimension_semantics=("parallel", "arbitrary"),
        ),
    )(w3, xp)
    if C_out_pad != C_out or L_pad != L_out:
        out = out[:, :C_out, :L_out]
    return out


# speedup vs baseline: 1.0908x; 1.0908x over previous
import functools

import jax
import jax.numpy as jnp
from jax.experimental import pallas as pl
from jax.experimental.pallas import tpu as pltpu

_CHUNKS = 8


def _round_up(v, m):
    return (v + m - 1) // m * m


def _conv3_kernel(w_ref, x_hbm, o_hbm, x_buf, o_buf, in_sem, out_sem,
                  *, rows, chunks):
    c_id = pl.program_id(0)
    row0 = c_id * rows
    l = x_buf.shape[2]
    t = l // chunks

    in_copy = pltpu.make_async_copy(
        x_hbm.at[pl.ds(row0, rows)], x_buf, in_sem)
    in_copy.start()
    in_copy.wait()

    out_copies = []
    for k in range(chunks):
        for r in range(rows):
            xc = x_buf[r, :, pl.ds(k * t, t)].astype(jnp.bfloat16)
            c = xc.shape[0]
            if k == 0:
                lcol = jnp.zeros((c, 1), jnp.bfloat16)
            else:
                lcol = x_buf[r, :, k * t - 1: k * t].astype(jnp.bfloat16)
            if k == chunks - 1:
                rcol = jnp.zeros((c, 1), jnp.bfloat16)
            else:
                rcol = x_buf[r, :, (k + 1) * t: (k + 1) * t + 1].astype(jnp.bfloat16)
            x3 = jnp.concatenate(
                [jnp.concatenate([lcol, xc[:, : t - 1]], axis=1),
                 xc,
                 jnp.concatenate([xc[:, 1:], rcol], axis=1)], axis=0)
            acc = jnp.dot(w_ref[...], x3, preferred_element_type=jnp.float32)
            o_buf[r, :, pl.ds(k * t, t)] = jnp.maximum(acc, 0.0)
        cp = pltpu.make_async_copy(
            o_buf.at[:, :, pl.ds(k * t, t)],
            o_hbm.at[pl.ds(row0, rows), :, pl.ds(k * t, t)],
            out_sem)
        cp.start()
        out_copies.append(cp)
    for cp in out_copies:
        cp.wait()


@jax.jit
def kernel(x, weight):
    N, C_in, L = x.shape
    C_out, C_in_w, K = weight.shape
    assert C_in_w == C_in and K == 3
    L_out = L

    C_in_pad = _round_up(C_in, 8)
    C_out_pad = _round_up(C_out, 8)
    L_pad = _round_up(L, 128 * _CHUNKS)
    xp = jnp.pad(x, ((0, 0), (0, C_in_pad - C_in), (0, L_pad - L)))
    w3 = jnp.transpose(weight, (2, 0, 1)).astype(jnp.bfloat16)
    w3 = jnp.pad(w3, ((0, 0), (0, C_out_pad - C_out), (0, C_in_pad - C_in)))
    w3 = jnp.transpose(w3, (1, 0, 2)).reshape(C_out_pad, K * C_in_pad)

    rows = N // 2
    out = pl.pallas_call(
        functools.partial(_conv3_kernel, rows=rows, chunks=_CHUNKS),
        out_shape=jax.ShapeDtypeStruct((N, C_out_pad, L_pad), x.dtype),
        grid=(2,),
        in_specs=[
            pl.BlockSpec((C_out_pad, K * C_in_pad), lambda c: (0, 0)),
            pl.BlockSpec(memory_space=pltpu.MemorySpace.HBM),
        ],
        out_specs=pl.BlockSpec(memory_space=pltpu.MemorySpace.HBM),
        scratch_shapes=[
            pltpu.VMEM((rows, C_in_pad, L_pad), jnp.float32),
            pltpu.VMEM((rows, C_out_pad, L_pad), jnp.float32),
            pltpu.SemaphoreType.DMA,
            pltpu.SemaphoreType.DMA,
        ],
        compiler_params=pltpu.CompilerParams(
            dimension_semantics=("parallel",),
            vmem_limit_bytes=60 * 1024 * 1024,
        ),
    )(w3, xp)
    if C_out_pad != C_out or L_pad != L_out:
        out = out[:, :C_out, :L_out]
    return out


# speedup vs baseline: 1.3971x; 1.2807x over previous
import functools

import jax
import jax.numpy as jnp
from jax.experimental import pallas as pl
from jax.experimental.pallas import tpu as pltpu

_CHUNKS = 8


def _round_up(v, m):
    return (v + m - 1) // m * m


def _conv3_kernel(w_ref, x_hbm, o_hbm, x_buf, o_buf, in_sem0, in_sem1, out_sem,
                  *, rows, chunks):
    del chunks
    c_id = pl.program_id(0)
    row0 = c_id * rows
    l = x_buf.shape[2]
    half = rows // 2 if rows >= 2 else rows

    in0 = pltpu.make_async_copy(
        x_hbm.at[pl.ds(row0, half)], x_buf.at[pl.ds(0, half)], in_sem0)
    in0.start()
    if rows >= 2:
        in1 = pltpu.make_async_copy(
            x_hbm.at[pl.ds(row0 + half, rows - half)],
            x_buf.at[pl.ds(half, rows - half)], in_sem1)
        in1.start()

    out_copies = []
    for r in range(rows):
        if r == 0:
            in0.wait()
        if rows >= 2 and r == half:
            in1.wait()
        xb = x_buf[r].astype(jnp.bfloat16)
        c = xb.shape[0]
        zero_col = jnp.zeros((c, 1), jnp.bfloat16)
        x3 = jnp.concatenate(
            [jnp.concatenate([zero_col, xb[:, : l - 1]], axis=1),
             xb,
             jnp.concatenate([xb[:, 1:], zero_col], axis=1)], axis=0)
        acc = jnp.dot(w_ref[...], x3, preferred_element_type=jnp.float32)
        o_buf[r] = jnp.maximum(acc, 0.0)
        cp = pltpu.make_async_copy(
            o_buf.at[pl.ds(r, 1)], o_hbm.at[pl.ds(row0 + r, 1)], out_sem)
        cp.start()
        out_copies.append(cp)
    for cp in out_copies:
        cp.wait()


@jax.jit
def kernel(x, weight):
    N, C_in, L = x.shape
    C_out, C_in_w, K = weight.shape
    assert C_in_w == C_in and K == 3
    L_out = L

    C_in_pad = _round_up(C_in, 8)
    C_out_pad = _round_up(C_out, 8)
    L_pad = _round_up(L, 128 * _CHUNKS)
    xp = jnp.pad(x, ((0, 0), (0, C_in_pad - C_in), (0, L_pad - L)))
    w3 = jnp.transpose(weight, (2, 0, 1)).astype(jnp.bfloat16)
    w3 = jnp.pad(w3, ((0, 0), (0, C_out_pad - C_out), (0, C_in_pad - C_in)))
    w3 = jnp.transpose(w3, (1, 0, 2)).reshape(C_out_pad, K * C_in_pad)

    rows = N // 2
    out = pl.pallas_call(
        functools.partial(_conv3_kernel, rows=rows, chunks=_CHUNKS),
        out_shape=jax.ShapeDtypeStruct((N, C_out_pad, L_pad), x.dtype),
        grid=(2,),
        in_specs=[
            pl.BlockSpec((C_out_pad, K * C_in_pad), lambda c: (0, 0)),
            pl.BlockSpec(memory_space=pltpu.MemorySpace.HBM),
        ],
        out_specs=pl.BlockSpec(memory_space=pltpu.MemorySpace.HBM),
        scratch_shapes=[
            pltpu.VMEM((rows, C_in_pad, L_pad), jnp.float32),
            pltpu.VMEM((rows, C_out_pad, L_pad), jnp.float32),
            pltpu.SemaphoreType.DMA,
            pltpu.SemaphoreType.DMA,
            pltpu.SemaphoreType.DMA,
        ],
        compiler_params=pltpu.CompilerParams(
            dimension_semantics=("parallel",),
            vmem_limit_bytes=60 * 1024 * 1024,
        ),
    )(w3, xp)
    if C_out_pad != C_out or L_pad != L_out:
        out = out[:, :C_out, :L_out]
    return out


# speedup vs baseline: 1.4262x; 1.0209x over previous
import functools

import jax
import jax.numpy as jnp
from jax.experimental import pallas as pl
from jax.experimental.pallas import tpu as pltpu

_GROUP = 2


def _round_up(v, m):
    return (v + m - 1) // m * m


def _conv3_kernel(w_ref, x_hbm, o_hbm, x_buf, o_buf, in_sem, out_sem,
                  *, rows, group):
    c_id = pl.program_id(0)
    row0 = c_id * rows
    l = x_buf.shape[2]
    g = group if rows % group == 0 else 1

    in_copies = []
    for i in range(0, rows, g):
        cp = pltpu.make_async_copy(
            x_hbm.at[pl.ds(row0 + i, g)], x_buf.at[pl.ds(i, g)], in_sem)
        cp.start()
        in_copies.append(cp)

    out_copies = []
    for r in range(rows):
        if r % g == 0:
            in_copies[r // g].wait()
        xb = x_buf[r].astype(jnp.bfloat16)
        c = xb.shape[0]
        zero_col = jnp.zeros((c, 1), jnp.bfloat16)
        x3 = jnp.concatenate(
            [jnp.concatenate([zero_col, xb[:, : l - 1]], axis=1),
             xb,
             jnp.concatenate([xb[:, 1:], zero_col], axis=1)], axis=0)
        acc = jnp.dot(w_ref[...], x3, preferred_element_type=jnp.float32)
        o_buf[r] = jnp.maximum(acc, 0.0)
        if r % g == g - 1:
            cp = pltpu.make_async_copy(
                o_buf.at[pl.ds(r - g + 1, g)],
                o_hbm.at[pl.ds(row0 + r - g + 1, g)], out_sem)
            cp.start()
            out_copies.append(cp)
    for cp in out_copies:
        cp.wait()


@jax.jit
def kernel(x, weight):
    N, C_in, L = x.shape
    C_out, C_in_w, K = weight.shape
    assert C_in_w == C_in and K == 3
    L_out = L

    C_in_pad = _round_up(C_in, 8)
    C_out_pad = _round_up(C_out, 8)
    L_pad = _round_up(L, 128)
    xp = jnp.pad(x, ((0, 0), (0, C_in_pad - C_in), (0, L_pad - L)))
    w3 = jnp.transpose(weight, (2, 0, 1)).astype(jnp.bfloat16)
    w3 = jnp.pad(w3, ((0, 0), (0, C_out_pad - C_out), (0, C_in_pad - C_in)))
    w3 = jnp.transpose(w3, (1, 0, 2)).reshape(C_out_pad, K * C_in_pad)

    rows = N // 2
    out = pl.pallas_call(
        functools.partial(_conv3_kernel, rows=rows, group=_GROUP),
        out_shape=jax.ShapeDtypeStruct((N, C_out_pad, L_pad), x.dtype),
        grid=(2,),
        in_specs=[
            pl.BlockSpec((C_out_pad, K * C_in_pad), lambda c: (0, 0)),
            pl.BlockSpec(memory_space=pltpu.MemorySpace.HBM),
        ],
        out_specs=pl.BlockSpec(memory_space=pltpu.MemorySpace.HBM),
        scratch_shapes=[
            pltpu.VMEM((rows, C_in_pad, L_pad), jnp.float32),
            pltpu.VMEM((rows, C_out_pad, L_pad), jnp.float32),
            pltpu.SemaphoreType.DMA,
            pltpu.SemaphoreType.DMA,
        ],
        compiler_params=pltpu.CompilerParams(
            dimension_semantics=("parallel",),
            vmem_limit_bytes=60 * 1024 * 1024,
        ),
    )(w3, xp)
    if C_out_pad != C_out or L_pad != L_out:
        out = out[:, :C_out, :L_out]
    return out
